# baseline (speedup 1.0000x reference)
import functools

import jax
import jax.numpy as jnp
from jax import lax
from jax.experimental import pallas as pl
from jax.experimental.pallas import tpu as pltpu
from jax.experimental.pallas import tpu_sc as plsc

N = 10000
E = 320000
B = 64
D_IN = 128
D_H = 64

NC = 2
NS = 16
NW = NC * NS
EWP = 10240
CK = 128
CHP = EWP // CK
RING = 5
IRING = 10
UNROLL = 10
TRIPS = CHP // UNROLL
NP = 10112
RPT = NP // NS
DW = 16

_mesh = plsc.VectorSubcoreMesh(core_axis_name="c", subcore_axis_name="s")
_sc_params = pltpu.CompilerParams(use_tc_tiling_on_sc=False)


@functools.partial(
    pl.kernel,
    mesh=_mesh,
    out_type=jax.ShapeDtypeStruct((NC, NP, DW), jnp.float32),
    scratch_types=[
        pltpu.VMEM((CHP, CK), jnp.int32),
        pltpu.VMEM((CK, DW), jnp.float32),
        pltpu.VMEM_SHARED((NP, DW), jnp.float32),
    ],
    compiler_params=_sc_params,
)
def _sc_degree(dst_hbm, zeros_hbm, ones_hbm, out_hbm, dst_v, ones_v, acc):
    cid = lax.axis_index("c")
    sid = lax.axis_index("s")
    wid = sid * NC + cid
    rows = pl.ds(sid * RPT, RPT)
    pltpu.sync_copy(zeros_hbm.at[rows], acc.at[rows])
    pltpu.sync_copy(dst_hbm.at[wid], dst_v)
    pltpu.sync_copy(ones_hbm, ones_v)
    plsc.subcore_barrier()

    def body(j, carry):
        pltpu.sync_copy(ones_v, acc.at[dst_v.at[j]], add=True)
        return carry

    lax.fori_loop(0, CHP, body, 0)
    plsc.subcore_barrier()
    pltpu.sync_copy(acc.at[rows], out_hbm.at[cid, rows, :])


@functools.partial(
    pl.kernel,
    mesh=_mesh,
    out_type=jax.ShapeDtypeStruct((NC, NP, D_H), jnp.float32),
    scratch_types=(
        [pltpu.VMEM((1, CK), jnp.int32) for _ in range(2 * IRING)]
        + [pltpu.VMEM((CK, D_H), jnp.float32) for _ in range(RING)]
        + [pltpu.SemaphoreType.DMA for _ in range(IRING + 2 * RING)]
        + [pltpu.VMEM_SHARED((NP, D_H), jnp.float32),
           pltpu.VMEM_SHARED((NP, D_H), jnp.float32)]
    ),
    compiler_params=_sc_params,
)
def _sc_propagate(hp_hbm, src_hbm, dst_hbm, zeros_hbm, out_hbm, *scr):
    isrc = scr[0:IRING]
    idst = scr[IRING:2 * IRING]
    rbuf = scr[2 * IRING:2 * IRING + RING]
    sem_i = scr[2 * IRING + RING:2 * IRING + RING + IRING]
    sem_g = scr[2 * IRING + RING + IRING:2 * IRING + 2 * RING + IRING]
    sem_s = scr[2 * IRING + 2 * RING + IRING:2 * IRING + 3 * RING + IRING]
    table = scr[-2]
    acc = scr[-1]
    cid = lax.axis_index("c")
    sid = lax.axis_index("s")
    wid = sid * NC + cid
    rows = pl.ds(sid * RPT, RPT)
    pltpu.sync_copy(zeros_hbm.at[rows], acc.at[rows])
    pltpu.sync_copy(hp_hbm.at[rows], table.at[rows])

    def fire_idx(c, slot, sem):
        pltpu.async_copy(src_hbm.at[wid, pl.ds(c, 1)], isrc[slot], sem)
        pltpu.async_copy(dst_hbm.at[wid, pl.ds(c, 1)], idst[slot], sem)

    def drain_idx(slot, sem):
        pltpu.make_async_copy(src_hbm.at[0, pl.ds(0, 1)], isrc[slot], sem).wait()
        pltpu.make_async_copy(src_hbm.at[0, pl.ds(0, 1)], idst[slot], sem).wait()

    def drain_row(buf, sem):
        pltpu.make_async_copy(hp_hbm.at[pl.ds(0, CK)], buf, sem).wait()

    def fire_gather(c, u):
        slot = u % RING
        isl = u % IRING
        if u % 2 == 0:
            pltpu.async_copy(hp_hbm.at[isrc[isl].at[0]], rbuf[slot],
                             sem_g[slot])
        else:
            pltpu.async_copy(table.at[isrc[isl].at[0]], rbuf[slot],
                             sem_g[slot])

    for u in range(4):
        fire_idx(u, u, sem_i[u])
    plsc.subcore_barrier()
    for u in range(2):
        drain_idx(u, sem_i[u])
        fire_gather(u, u)

    def body(it, carry):
        for u in range(UNROLL):
            c = it * UNROLL + u

            @pl.when(c >= 3)
            def _drain_s():
                drain_row(rbuf[(u + 2) % RING], sem_s[(u + 2) % RING])

            @pl.when(c + 4 < CHP)
            def _prefetch_idx():
                fire_idx(c + 4, (u + 4) % IRING, sem_i[(u + 4) % IRING])

            @pl.when(c + 2 < CHP)
            def _gather_next():
                drain_idx((u + 2) % IRING, sem_i[(u + 2) % IRING])
                fire_gather(c + 2, u + 2)

            drain_row(rbuf[u % RING], sem_g[u % RING])
            pltpu.async_copy(rbuf[u % RING], acc.at[idst[u % IRING].at[0]],
                             sem_s[u % RING], add=True)
        return carry

    lax.fori_loop(0, TRIPS, body, 0)
    for cc in (CHP - 3, CHP - 2, CHP - 1):
        drain_row(rbuf[cc % RING], sem_s[cc % RING])
    plsc.subcore_barrier()
    pltpu.sync_copy(acc.at[rows], out_hbm.at[cid, rows, :])


def _tc_mm_body(x_ref, w_ref, h_ref):
    h_ref[...] = jnp.dot(x_ref[...], w_ref[...],
                         preferred_element_type=jnp.float32)


def _tc_scale_body(h_ref, degp_ref, hp_ref, dis_ref):
    deg = degp_ref[0, :N, 0:1] + degp_ref[1, :N, 0:1] + 1.0
    dis = lax.rsqrt(deg)
    hp_ref[:N, :] = h_ref[...] * dis
    hp_ref[N:, :] = jnp.zeros((NP - N, D_H), jnp.float32)
    dis_ref[...] = dis


def _tc_mid_body(p_ref, hp_ref, dis_ref, b_ref, w_ref, out_ref):
    agg = p_ref[0, :N, :] + p_ref[1, :N, :] + hp_ref[:N, :]
    z = jnp.maximum(agg * dis_ref[...] + b_ref[...], 0.0)
    out_ref[:N, :] = (
        jnp.dot(z, w_ref[...], preferred_element_type=jnp.float32) * dis_ref[...]
    )
    out_ref[N:, :] = jnp.zeros((NP - N, D_H), jnp.float32)


def _tc_final_body(p_ref, hp_ref, dis_ref, b_ref, batch_ref, wp_ref, bp_ref,
                   out_ref):
    z = (p_ref[0, :N, :] + p_ref[1, :N, :] + hp_ref[:N, :]) * dis_ref[...] + b_ref[...]
    cols = lax.broadcasted_iota(jnp.int32, (1, B), 1)
    m = (batch_ref[...] == cols).astype(jnp.float32)
    cdims = (((0,), (0,)), ((), ()))
    sums = lax.dot_general(m, z, cdims, preferred_element_type=jnp.float32)
    counts = lax.dot_general(m, jnp.ones((N, 1), jnp.float32), cdims,
                             preferred_element_type=jnp.float32)
    pooled = sums / jnp.maximum(counts, 1.0)
    out_ref[...] = (
        jnp.dot(pooled, wp_ref[...], preferred_element_type=jnp.float32)
        + bp_ref[...]
    )


_tc_mm = pl.pallas_call(
    _tc_mm_body,
    out_shape=jax.ShapeDtypeStruct((N, D_H), jnp.float32),
)

_tc_scale = pl.pallas_call(
    _tc_scale_body,
    out_shape=[
        jax.ShapeDtypeStruct((NP, D_H), jnp.float32),
        jax.ShapeDtypeStruct((N, 1), jnp.float32),
    ],
)

_tc_mid = pl.pallas_call(
    _tc_mid_body,
    out_shape=jax.ShapeDtypeStruct((NP, D_H), jnp.float32),
)

_tc_final = pl.pallas_call(
    _tc_final_body,
    out_shape=jax.ShapeDtypeStruct((B, 1), jnp.float32),
)


def kernel(x, edge_index, batch, W1, b1, W2, b2, W3, b3, Wp, bp):
    pad = NW * EWP - E
    pidx = jnp.arange(pad, dtype=jnp.int32)
    src_pad = (pidx * 131) % N
    dst_pad = N + pidx % (NP - N)
    src_r = jnp.concatenate([edge_index[0], src_pad]).reshape(NW, CHP, CK)
    dst_r = jnp.concatenate([edge_index[1], dst_pad]).reshape(NW, CHP, CK)
    zeros = jnp.zeros((NP, D_H), jnp.float32)
    zeros_dw = jnp.zeros((NP, DW), jnp.float32)
    ones_dw = jnp.ones((CK, DW), jnp.float32)
    batch2 = batch.reshape(N, 1)
    b1r = b1.reshape(1, D_H)
    b2r = b2.reshape(1, D_H)
    b3r = b3.reshape(1, D_H)
    bpr = bp.reshape(1, 1)

    degp = _sc_degree(dst_r, zeros_dw, ones_dw)
    h1 = _tc_mm(x, W1)
    hp1, dis = _tc_scale(h1, degp)
    p1 = _sc_propagate(hp1, src_r, dst_r, zeros)
    hp2 = _tc_mid(p1, hp1, dis, b1r, W2)
    p2 = _sc_propagate(hp2, src_r, dst_r, zeros)
    hp3 = _tc_mid(p2, hp2, dis, b2r, W3)
    p3 = _sc_propagate(hp3, src_r, dst_r, zeros)
    return _tc_final(p3, hp3, dis, b3r, batch2, Wp, bpr)


# speedup vs baseline: 1.3124x; 1.3124x over previous
import functools

import jax
import jax.numpy as jnp
from jax import lax
from jax.experimental import pallas as pl
from jax.experimental.pallas import tpu as pltpu
from jax.experimental.pallas import tpu_sc as plsc

N = 10000
E = 320000
B = 64
D_IN = 128
D_H = 64

NC = 2
NS = 16
NW = NC * NS
EWP = 10240
CK = 128
CHP = EWP // CK
K = 4
P = CHP // (2 * K)
NP = 10240
RPT = NP // NS
DW = 16

_mesh = plsc.VectorSubcoreMesh(core_axis_name="c", subcore_axis_name="s")
_sc_params = pltpu.CompilerParams(use_tc_tiling_on_sc=False)


@functools.partial(
    pl.kernel,
    mesh=_mesh,
    out_type=jax.ShapeDtypeStruct((NC, NP, DW), jnp.float32),
    scratch_types=[
        pltpu.VMEM((CHP, CK), jnp.int32),
        pltpu.VMEM((CK, DW), jnp.float32),
        pltpu.SemaphoreType.DMA,
        pltpu.VMEM_SHARED((NP, DW), jnp.float32),
    ],
    compiler_params=_sc_params,
)
def _sc_degree(dst_hbm, zeros_hbm, ones_hbm, out_hbm, dst_v, ones_v, sem_d,
               acc):
    cid = lax.axis_index("c")
    sid = lax.axis_index("s")
    wid = sid * NC + cid
    rows = pl.ds(sid * RPT, RPT)
    pltpu.sync_copy(zeros_hbm.at[rows], acc.at[rows])
    pltpu.sync_copy(dst_hbm.at[wid], dst_v)
    pltpu.sync_copy(ones_hbm, ones_v)
    plsc.subcore_barrier()

    def body(j, carry):
        pltpu.async_copy(ones_v, acc.at[dst_v.at[j]], sem_d, add=True)

        @pl.when(j >= 8)
        def _drain_one():
            pltpu.make_async_copy(ones_hbm, ones_v, sem_d).wait()

        return carry

    lax.fori_loop(0, CHP, body, 0)

    def drain_tail(j, carry):
        pltpu.make_async_copy(ones_hbm, ones_v, sem_d).wait()
        return carry

    lax.fori_loop(0, 8, drain_tail, 0)
    plsc.subcore_barrier()
    pltpu.sync_copy(acc.at[rows], out_hbm.at[cid, rows, :])


@functools.partial(
    pl.kernel,
    mesh=_mesh,
    out_type=jax.ShapeDtypeStruct((NC, NP, D_H), jnp.float32),
    scratch_types=[
        pltpu.VMEM((CHP, CK), jnp.int32),
        pltpu.VMEM((CHP, CK), jnp.int32),
    ]
    + [pltpu.VMEM((CK, D_H), jnp.float32) for _ in range(8)]
    + [pltpu.SemaphoreType.DMA for _ in range(4)]
    + [pltpu.VMEM_SHARED((NP, D_H), jnp.float32)],
    compiler_params=_sc_params,
)
def _sc_propagate(hp_hbm, src_hbm, dst_hbm, zeros_hbm, out_hbm,
                  src_v, dst_v,
                  ra0, ra1, ra2, ra3, rb0, rb1, rb2, rb3,
                  sem_ga, sem_sa, sem_gb, sem_sb, acc):
    ra = (ra0, ra1, ra2, ra3)
    rb = (rb0, rb1, rb2, rb3)
    cid = lax.axis_index("c")
    sid = lax.axis_index("s")
    wid = sid * NC + cid
    rows = pl.ds(sid * RPT, RPT)
    pltpu.sync_copy(src_hbm.at[wid], src_v)
    for b in range(K):
        pltpu.async_copy(hp_hbm.at[src_v.at[b]], ra[b], sem_ga)
    pltpu.sync_copy(zeros_hbm.at[rows], acc.at[rows])
    pltpu.sync_copy(dst_hbm.at[wid], dst_v)
    plsc.subcore_barrier()

    def drain(buf, sem):
        pltpu.make_async_copy(hp_hbm.at[pl.ds(0, CK)], buf, sem).wait()

    def body(p, carry):
        base = p * 2 * K
        for b in range(K):
            drain(ra[b], sem_ga)
        for b in range(K):
            pltpu.async_copy(ra[b], acc.at[dst_v.at[base + b]], sem_sa,
                             add=True)

        @pl.when(p > 0)
        def _wait_prev_b():
            for b in range(K):
                drain(rb[b], sem_sb)

        for b in range(K):
            pltpu.async_copy(hp_hbm.at[src_v.at[base + K + b]], rb[b], sem_gb)
        for b in range(K):
            drain(rb[b], sem_gb)
        for b in range(K):
            pltpu.async_copy(rb[b], acc.at[dst_v.at[base + K + b]], sem_sb,
                             add=True)

        @pl.when(p < P - 1)
        def _refill_a():
            for b in range(K):
                drain(ra[b], sem_sa)
            for b in range(K):
                pltpu.async_copy(hp_hbm.at[src_v.at[base + 2 * K + b]], ra[b],
                                 sem_ga)

        return carry

    lax.fori_loop(0, P, body, 0)
    for b in range(K):
        drain(ra[b], sem_sa)
    for b in range(K):
        drain(rb[b], sem_sb)
    plsc.subcore_barrier()
    pltpu.sync_copy(acc.at[rows], out_hbm.at[cid, rows, :])


def _tc_first_body(x_ref, w_ref, degp_ref, hp_ref, dis_ref):
    deg = degp_ref[0, :N, 0:1] + degp_ref[1, :N, 0:1] + 1.0
    dis = lax.rsqrt(deg)
    h = jnp.dot(x_ref[...], w_ref[...], preferred_element_type=jnp.float32)
    hp_ref[...] = h * dis
    dis_ref[...] = dis


def _tc_mid_body(p_ref, hp_ref, dis_ref, b_ref, w_ref, out_ref):
    agg = p_ref[0, :N, :] + p_ref[1, :N, :] + hp_ref[...]
    z = jnp.maximum(agg * dis_ref[...] + b_ref[...], 0.0)
    out_ref[...] = (
        jnp.dot(z, w_ref[...], preferred_element_type=jnp.float32) * dis_ref[...]
    )


def _tc_final_body(p_ref, hp_ref, dis_ref, b_ref, batch_ref, wp_ref, bp_ref,
                   out_ref):
    z = (p_ref[0, :N, :] + p_ref[1, :N, :] + hp_ref[...]) * dis_ref[...] + b_ref[...]
    cols = lax.broadcasted_iota(jnp.int32, (1, B), 1)
    m = (batch_ref[...] == cols).astype(jnp.float32)
    cdims = (((0,), (0,)), ((), ()))
    sums = lax.dot_general(m, z, cdims, preferred_element_type=jnp.float32)
    counts = lax.dot_general(m, jnp.ones((N, 1), jnp.float32), cdims,
                             preferred_element_type=jnp.float32)
    pooled = sums / jnp.maximum(counts, 1.0)
    out_ref[...] = (
        jnp.dot(pooled, wp_ref[...], preferred_element_type=jnp.float32)
        + bp_ref[...]
    )


_tc_first = pl.pallas_call(
    _tc_first_body,
    out_shape=[
        jax.ShapeDtypeStruct((N, D_H), jnp.float32),
        jax.ShapeDtypeStruct((N, 1), jnp.float32),
    ],
)

_tc_mid = pl.pallas_call(
    _tc_mid_body,
    out_shape=jax.ShapeDtypeStruct((N, D_H), jnp.float32),
)

_tc_final = pl.pallas_call(
    _tc_final_body,
    out_shape=jax.ShapeDtypeStruct((B, 1), jnp.float32),
)


def kernel(x, edge_index, batch, W1, b1, W2, b2, W3, b3, Wp, bp):
    pad = NW * EWP - E
    pidx = jnp.arange(pad, dtype=jnp.int32)
    src_pad = (pidx * 131) % N
    dst_pad = N + pidx % (NP - N)
    src_r = jnp.concatenate([edge_index[0], src_pad]).reshape(NW, CHP, CK)
    dst_r = jnp.concatenate([edge_index[1], dst_pad]).reshape(NW, CHP, CK)
    zeros = jnp.zeros((NP, D_H), jnp.float32)
    zeros_dw = jnp.zeros((NP, DW), jnp.float32)
    ones_dw = jnp.ones((CK, DW), jnp.float32)
    batch2 = batch.reshape(N, 1)
    b1r = b1.reshape(1, D_H)
    b2r = b2.reshape(1, D_H)
    b3r = b3.reshape(1, D_H)
    bpr = bp.reshape(1, 1)

    degp = _sc_degree(dst_r, zeros_dw, ones_dw)
    hp1, dis = _tc_first(x, W1, degp)
    p1 = _sc_propagate(hp1, src_r, dst_r, zeros)
    hp2 = _tc_mid(p1, hp1, dis, b1r, W2)
    p2 = _sc_propagate(hp2, src_r, dst_r, zeros)
    hp3 = _tc_mid(p2, hp2, dis, b2r, W3)
    p3 = _sc_propagate(hp3, src_r, dst_r, zeros)
    return _tc_final(p3, hp3, dis, b3r, batch2, Wp, bpr)


# speedup vs baseline: 1.3262x; 1.0106x over previous
import functools

import jax
import jax.numpy as jnp
from jax import lax
from jax.experimental import pallas as pl
from jax.experimental.pallas import tpu as pltpu
from jax.experimental.pallas import tpu_sc as plsc

N = 10000
E = 320000
B = 64
D_IN = 128
D_H = 64

NC = 2
NS = 16
NW = NC * NS
EWP = 10240
CK = 128
CHP = EWP // CK
K = 4
P = CHP // (2 * K)
NP = 10240
RPT = NP // NS
DW = 8

_mesh = plsc.VectorSubcoreMesh(core_axis_name="c", subcore_axis_name="s")
_sc_params = pltpu.CompilerParams(use_tc_tiling_on_sc=False)


@functools.partial(
    pl.kernel,
    mesh=_mesh,
    out_type=jax.ShapeDtypeStruct((NC, NP, DW), jnp.float32),
    scratch_types=[
        pltpu.VMEM((CHP, CK), jnp.int32),
        pltpu.VMEM((CK, DW), jnp.float32),
        pltpu.SemaphoreType.DMA,
        pltpu.VMEM_SHARED((NP, DW), jnp.float32),
    ],
    compiler_params=_sc_params,
)
def _sc_degree(dst_hbm, zeros_hbm, ones_hbm, out_hbm, dst_v, ones_v, sem_d,
               acc):
    cid = lax.axis_index("c")
    sid = lax.axis_index("s")
    wid = sid * NC + cid
    rows = pl.ds(sid * RPT, RPT)
    pltpu.sync_copy(zeros_hbm.at[rows], acc.at[rows])
    pltpu.sync_copy(dst_hbm.at[wid], dst_v)
    pltpu.sync_copy(ones_hbm, ones_v)
    plsc.subcore_barrier()

    def body(j, carry):
        pltpu.async_copy(ones_v, acc.at[dst_v.at[j]], sem_d, add=True)

        @pl.when(j >= 8)
        def _drain_one():
            pltpu.make_async_copy(ones_hbm, ones_v, sem_d).wait()

        return carry

    lax.fori_loop(0, CHP, body, 0)

    def drain_tail(j, carry):
        pltpu.make_async_copy(ones_hbm, ones_v, sem_d).wait()
        return carry

    lax.fori_loop(0, 8, drain_tail, 0)
    plsc.subcore_barrier()
    pltpu.sync_copy(acc.at[rows], out_hbm.at[cid, rows, :])


@functools.partial(
    pl.kernel,
    mesh=_mesh,
    out_type=jax.ShapeDtypeStruct((NC, NP, D_H), jnp.float32),
    scratch_types=[
        pltpu.VMEM((CHP, CK), jnp.int32),
        pltpu.VMEM((CHP, CK), jnp.int32),
    ]
    + [pltpu.VMEM((CK, D_H), jnp.float32) for _ in range(8)]
    + [pltpu.SemaphoreType.DMA for _ in range(4)]
    + [pltpu.VMEM_SHARED((NP, D_H), jnp.float32)],
    compiler_params=_sc_params,
)
def _sc_propagate(hp_hbm, src_hbm, dst_hbm, zeros_hbm, out_hbm,
                  src_v, dst_v,
                  ra0, ra1, ra2, ra3, rb0, rb1, rb2, rb3,
                  sem_ga, sem_sa, sem_gb, sem_sb, acc):
    ra = (ra0, ra1, ra2, ra3)
    rb = (rb0, rb1, rb2, rb3)
    cid = lax.axis_index("c")
    sid = lax.axis_index("s")
    wid = sid * NC + cid
    rows = pl.ds(sid * RPT, RPT)
    pltpu.sync_copy(src_hbm.at[wid], src_v)
    for b in range(K):
        pltpu.async_copy(hp_hbm.at[src_v.at[b]], ra[b], sem_ga)
    pltpu.sync_copy(zeros_hbm.at[rows], acc.at[rows])
    pltpu.sync_copy(dst_hbm.at[wid], dst_v)
    plsc.subcore_barrier()

    def drain(buf, sem):
        pltpu.make_async_copy(hp_hbm.at[pl.ds(0, CK)], buf, sem).wait()

    def body(p, carry):
        base = p * 2 * K
        for b in range(K):
            drain(ra[b], sem_ga)
        for b in range(K):
            pltpu.async_copy(ra[b], acc.at[dst_v.at[base + b]], sem_sa,
                             add=True)

        @pl.when(p > 0)
        def _wait_prev_b():
            for b in range(K):
                drain(rb[b], sem_sb)

        for b in range(K):
            pltpu.async_copy(hp_hbm.at[src_v.at[base + K + b]], rb[b], sem_gb)
        for b in range(K):
            drain(rb[b], sem_gb)
        for b in range(K):
            pltpu.async_copy(rb[b], acc.at[dst_v.at[base + K + b]], sem_sb,
                             add=True)

        @pl.when(p < P - 1)
        def _refill_a():
            for b in range(K):
                drain(ra[b], sem_sa)
            for b in range(K):
                pltpu.async_copy(hp_hbm.at[src_v.at[base + 2 * K + b]], ra[b],
                                 sem_ga)

        return carry

    lax.fori_loop(0, P, body, 0)
    for b in range(K):
        drain(ra[b], sem_sa)
    for b in range(K):
        drain(rb[b], sem_sb)
    plsc.subcore_barrier()
    pltpu.sync_copy(acc.at[rows], out_hbm.at[cid, rows, :])


def _tc_first_body(x_ref, w_ref, degp_ref, hp_ref, dis_ref):
    deg = degp_ref[0, :N, 0:1] + degp_ref[1, :N, 0:1] + 1.0
    dis = lax.rsqrt(deg)
    h = jnp.dot(x_ref[...], w_ref[...], preferred_element_type=jnp.float32)
    hp_ref[...] = h * dis
    dis_ref[...] = dis


def _tc_mid_body(p_ref, hp_ref, dis_ref, b_ref, w_ref, out_ref):
    agg = p_ref[0, :N, :] + p_ref[1, :N, :] + hp_ref[...]
    z = jnp.maximum(agg * dis_ref[...] + b_ref[...], 0.0)
    out_ref[...] = (
        jnp.dot(z, w_ref[...], preferred_element_type=jnp.float32) * dis_ref[...]
    )


def _tc_final_body(p_ref, hp_ref, dis_ref, b_ref, batch_ref, wp_ref, bp_ref,
                   out_ref):
    z = (p_ref[0, :N, :] + p_ref[1, :N, :] + hp_ref[...]) * dis_ref[...] + b_ref[...]
    cols = lax.broadcasted_iota(jnp.int32, (1, B), 1)
    m = (batch_ref[...] == cols).astype(jnp.float32)
    cdims = (((0,), (0,)), ((), ()))
    sums = lax.dot_general(m, z, cdims, preferred_element_type=jnp.float32)
    counts = lax.dot_general(m, jnp.ones((N, 1), jnp.float32), cdims,
                             preferred_element_type=jnp.float32)
    pooled = sums / jnp.maximum(counts, 1.0)
    out_ref[...] = (
        jnp.dot(pooled, wp_ref[...], preferred_element_type=jnp.float32)
        + bp_ref[...]
    )


_tc_first = pl.pallas_call(
    _tc_first_body,
    out_shape=[
        jax.ShapeDtypeStruct((N, D_H), jnp.float32),
        jax.ShapeDtypeStruct((N, 1), jnp.float32),
    ],
)

_tc_mid = pl.pallas_call(
    _tc_mid_body,
    out_shape=jax.ShapeDtypeStruct((N, D_H), jnp.float32),
)

_tc_final = pl.pallas_call(
    _tc_final_body,
    out_shape=jax.ShapeDtypeStruct((B, 1), jnp.float32),
)


def kernel(x, edge_index, batch, W1, b1, W2, b2, W3, b3, Wp, bp):
    pad = NW * EWP - E
    pidx = jnp.arange(pad, dtype=jnp.int32)
    src_pad = (pidx * 131) % N
    dst_pad = N + pidx % (NP - N)
    src_r = jnp.concatenate([edge_index[0], src_pad]).reshape(NW, CHP, CK)
    dst_r = jnp.concatenate([edge_index[1], dst_pad]).reshape(NW, CHP, CK)
    zeros = jnp.zeros((NP, D_H), jnp.float32)
    zeros_dw = jnp.zeros((NP, DW), jnp.float32)
    ones_dw = jnp.ones((CK, DW), jnp.float32)
    batch2 = batch.reshape(N, 1)
    b1r = b1.reshape(1, D_H)
    b2r = b2.reshape(1, D_H)
    b3r = b3.reshape(1, D_H)
    bpr = bp.reshape(1, 1)

    degp = _sc_degree(dst_r, zeros_dw, ones_dw)
    hp1, dis = _tc_first(x, W1, degp)
    p1 = _sc_propagate(hp1, src_r, dst_r, zeros)
    hp2 = _tc_mid(p1, hp1, dis, b1r, W2)
    p2 = _sc_propagate(hp2, src_r, dst_r, zeros)
    hp3 = _tc_mid(p2, hp2, dis, b2r, W3)
    p3 = _sc_propagate(hp3, src_r, dst_r, zeros)
    return _tc_final(p3, hp3, dis, b3r, batch2, Wp, bpr)
